# Initial kernel scaffold; baseline (speedup 1.0000x reference)
import jax
import jax.numpy as jnp
from jax.experimental import pallas as pl


def kernel(input, embedding_matrix):
    raise NotImplementedError("write your pallas kernel here")


# speedup vs baseline: 1.0954x; 1.0954x over previous
import functools

import jax
import jax.numpy as jnp
from jax import lax
from jax.experimental import pallas as pl
from jax.experimental.pallas import tpu as pltpu
from jax.experimental.pallas import tpu_sc as plsc

_LANES_PER_ROW = 128
_G = 8


@functools.partial(jax.jit, static_argnums=(2, 3, 4))
def _sc_gather(idx2, table, n_workers, rows_per_worker, d):
    groups = rows_per_worker // _G
    chunk = _G * _LANES_PER_ROW

    mesh = plsc.VectorSubcoreMesh(core_axis_name="c", subcore_axis_name="s")

    @functools.partial(
        pl.kernel,
        out_type=jax.ShapeDtypeStruct((idx2.shape[0] * _LANES_PER_ROW, d),
                                      jnp.float32),
        mesh=mesh,
        scratch_types=[
            pltpu.VMEM((_G, _LANES_PER_ROW), jnp.int32),
            pltpu.VMEM((chunk, d), jnp.float32),
            pltpu.SemaphoreType.DMA,
        ],
        compiler_params=pltpu.CompilerParams(use_tc_tiling_on_sc=False),
    )
    def k(idx_hbm, table_hbm, out_hbm, idx_v, rows_v, sem):
        nc = lax.axis_size("c")
        wid = lax.axis_index("s") * nc + lax.axis_index("c")
        row0 = wid * rows_per_worker
        out0 = row0 * _LANES_PER_ROW

        def body(g, carry):
            pltpu.sync_copy(idx_hbm.at[pl.ds(row0 + g * _G, _G)], idx_v)
            copies = [
                pltpu.async_copy(
                    table_hbm.at[idx_v.at[j]],
                    rows_v.at[pl.ds(j * _LANES_PER_ROW, _LANES_PER_ROW)],
                    sem,
                )
                for j in range(_G)
            ]
            for c in copies:
                c.wait()
            pltpu.sync_copy(rows_v, out_hbm.at[pl.ds(out0 + g * chunk, chunk)])
            return carry

        lax.fori_loop(0, groups, body, 0)

    return k(idx2, table)


def kernel(input, embedding_matrix):
    b, h = input.shape
    v, d = embedding_matrix.shape
    n = b * h

    info = plsc.get_sparse_core_info()
    n_workers = info.num_cores * info.num_subcores

    assert n % (n_workers * _G * _LANES_PER_ROW) == 0
    rows_per_worker = n // (n_workers * _LANES_PER_ROW)

    idx2 = input.reshape(n // _LANES_PER_ROW, _LANES_PER_ROW).astype(jnp.int32)
    out = _sc_gather(idx2, embedding_matrix, n_workers, rows_per_worker, d)
    return out.reshape(b, h, d)


# speedup vs baseline: 1.1059x; 1.0096x over previous
import functools

import jax
import jax.numpy as jnp
from jax import lax
from jax.experimental import pallas as pl
from jax.experimental.pallas import tpu as pltpu
from jax.experimental.pallas import tpu_sc as plsc

_LANES_PER_ROW = 128
_G = 8


@functools.partial(jax.jit, static_argnums=(2, 3, 4))
def _sc_gather(idx2, table, n_workers, rows_per_worker, d):
    groups = rows_per_worker // _G
    assert groups % 2 == 1 and groups >= 5
    chunk = _G * _LANES_PER_ROW

    mesh = plsc.VectorSubcoreMesh(core_axis_name="c", subcore_axis_name="s")

    @functools.partial(
        pl.kernel,
        out_type=jax.ShapeDtypeStruct((idx2.shape[0] * _LANES_PER_ROW, d),
                                      jnp.float32),
        mesh=mesh,
        scratch_types=[
            pltpu.VMEM((_G, _LANES_PER_ROW), jnp.int32),
            pltpu.VMEM((_G, _LANES_PER_ROW), jnp.int32),
            pltpu.VMEM((chunk, d), jnp.float32),
            pltpu.VMEM((chunk, d), jnp.float32),
            pltpu.SemaphoreType.DMA,
            pltpu.SemaphoreType.DMA,
            pltpu.SemaphoreType.DMA,
            pltpu.SemaphoreType.DMA,
        ],
        compiler_params=pltpu.CompilerParams(use_tc_tiling_on_sc=False),
    )
    def k(idx_hbm, table_hbm, out_hbm, idx0, idx1, rows0, rows1,
          gsem0, gsem1, ssem0, ssem1):
        idxv = (idx0, idx1)
        rows = (rows0, rows1)
        gsem = (gsem0, gsem1)
        ssem = (ssem0, ssem1)

        nc = lax.axis_size("c")
        wid = lax.axis_index("s") * nc + lax.axis_index("c")
        row0 = wid * rows_per_worker
        out0 = row0 * _LANES_PER_ROW

        def fire_gathers(g, b):
            pltpu.sync_copy(idx_hbm.at[pl.ds(row0 + g * _G, _G)], idxv[b])
            for j in range(_G):
                pltpu.async_copy(
                    table_hbm.at[idxv[b].at[j]],
                    rows[b].at[pl.ds(j * _LANES_PER_ROW, _LANES_PER_ROW)],
                    gsem[b],
                )

        def drain_gathers(b):
            for j in range(_G):
                pltpu.make_async_copy(
                    table_hbm.at[idxv[b].at[j]],
                    rows[b].at[pl.ds(j * _LANES_PER_ROW, _LANES_PER_ROW)],
                    gsem[b],
                ).wait()

        def fire_store(g, b):
            pltpu.async_copy(
                rows[b], out_hbm.at[pl.ds(out0 + g * chunk, chunk)], ssem[b])

        def drain_store(g, b):
            pltpu.make_async_copy(
                rows[b], out_hbm.at[pl.ds(out0 + g * chunk, chunk)], ssem[b]
            ).wait()

        fire_gathers(0, 0)
        fire_gathers(1, 1)
        drain_gathers(0)
        fire_store(0, 0)

        def body(j, carry):
            g = 2 * j + 1
            drain_store(g - 1, 0)
            fire_gathers(g + 1, 0)
            drain_gathers(1)
            fire_store(g, 1)

            drain_store(g, 1)
            fire_gathers(g + 2, 1)
            drain_gathers(0)
            fire_store(g + 1, 0)
            return carry

        lax.fori_loop(0, (groups - 3) // 2, body, 0)

        drain_store(groups - 3, 0)
        fire_gathers(groups - 1, 0)
        drain_gathers(1)
        fire_store(groups - 2, 1)
        drain_gathers(0)
        fire_store(groups - 1, 0)
        drain_store(groups - 2, 1)
        drain_store(groups - 1, 0)

    return k(idx2, table)


def kernel(input, embedding_matrix):
    b, h = input.shape
    v, d = embedding_matrix.shape
    n = b * h

    info = plsc.get_sparse_core_info()
    n_workers = info.num_cores * info.num_subcores

    assert n % (n_workers * _G * _LANES_PER_ROW) == 0
    rows_per_worker = n // (n_workers * _LANES_PER_ROW)

    idx2 = input.reshape(n // _LANES_PER_ROW, _LANES_PER_ROW).astype(jnp.int32)
    out = _sc_gather(idx2, embedding_matrix, n_workers, rows_per_worker, d)
    return out.reshape(b, h, d)


# speedup vs baseline: 1.7575x; 1.5892x over previous
import functools

import jax
import jax.numpy as jnp
from jax import lax
from jax.experimental import pallas as pl
from jax.experimental.pallas import tpu as pltpu
from jax.experimental.pallas import tpu_sc as plsc

_G = 8


@functools.partial(jax.jit, static_argnums=(2,))
def _sc_gather(idx, table, n_workers):
    b, h = idx.shape
    _, d = table.shape
    b_per_w = b // n_workers
    groups = b_per_w // _G
    assert groups % 2 == 0 and groups >= 4

    mesh = plsc.VectorSubcoreMesh(core_axis_name="c", subcore_axis_name="s")

    @functools.partial(
        pl.kernel,
        out_type=jax.ShapeDtypeStruct((b, h, d), jnp.float32),
        mesh=mesh,
        scratch_types=[
            pltpu.VMEM((_G, h), jnp.int32),
            pltpu.VMEM((_G, h), jnp.int32),
            pltpu.VMEM((_G, h, d), jnp.float32),
            pltpu.VMEM((_G, h, d), jnp.float32),
            pltpu.SemaphoreType.DMA,
            pltpu.SemaphoreType.DMA,
            pltpu.SemaphoreType.DMA,
            pltpu.SemaphoreType.DMA,
        ],
        compiler_params=pltpu.CompilerParams(use_tc_tiling_on_sc=False),
    )
    def k(idx_hbm, table_hbm, out_hbm, idx0, idx1, rows0, rows1,
          gsem0, gsem1, ssem0, ssem1):
        idxv = (idx0, idx1)
        rows = (rows0, rows1)
        gsem = (gsem0, gsem1)
        ssem = (ssem0, ssem1)

        nc = lax.axis_size("c")
        wid = lax.axis_index("s") * nc + lax.axis_index("c")
        b0 = wid * b_per_w

        def fire_gathers(g, buf):
            pltpu.sync_copy(idx_hbm.at[pl.ds(b0 + g * _G, _G)], idxv[buf])
            for j in range(_G):
                pltpu.async_copy(
                    table_hbm.at[idxv[buf].at[j]],
                    rows[buf].at[j],
                    gsem[buf],
                )

        def drain_gathers(buf):
            for j in range(_G):
                pltpu.make_async_copy(
                    table_hbm.at[idxv[buf].at[j]],
                    rows[buf].at[j],
                    gsem[buf],
                ).wait()

        def fire_store(g, buf):
            pltpu.async_copy(
                rows[buf], out_hbm.at[pl.ds(b0 + g * _G, _G)], ssem[buf])

        def drain_store(g, buf):
            pltpu.make_async_copy(
                rows[buf], out_hbm.at[pl.ds(b0 + g * _G, _G)], ssem[buf]
            ).wait()

        fire_gathers(0, 0)
        fire_gathers(1, 1)
        drain_gathers(0)
        fire_store(0, 0)

        def body(j, carry):
            g = 2 * j + 1
            drain_store(g - 1, 0)
            fire_gathers(g + 1, 0)
            drain_gathers(1)
            fire_store(g, 1)

            drain_store(g, 1)
            fire_gathers(g + 2, 1)
            drain_gathers(0)
            fire_store(g + 1, 0)
            return carry

        lax.fori_loop(0, (groups - 2) // 2, body, 0)

        drain_gathers(1)
        fire_store(groups - 1, 1)
        drain_store(groups - 2, 0)
        drain_store(groups - 1, 1)

    return k(idx, table)


def kernel(input, embedding_matrix):
    b, h = input.shape

    info = plsc.get_sparse_core_info()
    n_workers = info.num_cores * info.num_subcores

    assert b % (n_workers * _G) == 0
    return _sc_gather(input.astype(jnp.int32), embedding_matrix, n_workers)
